# baseline (speedup 1.0000x reference)
import functools

import jax
import jax.numpy as jnp
from jax import lax
from jax.experimental import pallas as pl
from jax.experimental.pallas import tpu as pltpu
from jax.experimental.pallas import tpu_sc as plsc

_N = 50000
_E = 800000
_OUT = 64
_HALF = 32
_T = 4
_STEPS = 8
_NCORES = 2
_NTILES = 16
_CHUNK = 128
_KB = 4
_CPT = 392
_G = _CPT // _KB
_NCH = _CPT * _NTILES
_EP = _NCH * _CHUNK
_NP = 50048
_RPT = _NP // _NTILES
_NB = 2000
_GRID = _N // _NB
_K8 = _NCORES * _T


def _sc_aggregate(ht_flat, gidx2, sdst2, zeros):
    mesh = plsc.VectorSubcoreMesh(
        core_axis_name="c", subcore_axis_name="s",
        num_cores=_NCORES, num_subcores=_NTILES)

    @functools.partial(
        pl.kernel,
        out_type=jax.ShapeDtypeStruct((_NCORES, _NP, _HALF), jnp.float32),
        mesh=mesh,
        scratch_types=[
            pltpu.VMEM((_KB, _CHUNK), jnp.int32),
            pltpu.VMEM((_KB, _CHUNK), jnp.int32),
            pltpu.VMEM((_KB, _CHUNK, _HALF), jnp.float32),
            pltpu.VMEM_SHARED((_NP, _HALF), jnp.float32),
            pltpu.SemaphoreType.DMA,
        ],
        compiler_params=pltpu.CompilerParams(use_tc_tiling_on_sc=False),
    )
    def k(ht_hbm, gidx_hbm, sdst_hbm, zeros_hbm, out_hbm,
          gidx_v, sidx_v, rows_v, acc, sem):
        c = lax.axis_index("c")
        s = lax.axis_index("s")
        r0 = s * _RPT
        pltpu.sync_copy(zeros_hbm.at[pl.ds(r0, _RPT)], acc.at[pl.ds(r0, _RPT)])
        plsc.subcore_barrier()

        def group(g, carry):
            base = s * _CPT + g * _KB
            pltpu.sync_copy(gidx_hbm.at[c, pl.ds(base, _KB)], gidx_v)
            pltpu.sync_copy(sdst_hbm.at[pl.ds(base, _KB)], sidx_v)
            descs = [pltpu.async_copy(ht_hbm.at[gidx_v.at[j]], rows_v.at[j], sem)
                     for j in range(_KB)]
            for j in range(_KB):
                descs[j].wait()
                pltpu.sync_copy(rows_v.at[j], acc.at[sidx_v.at[j]], add=True)
            return carry

        lax.fori_loop(0, _G, group, 0)
        plsc.subcore_barrier()
        pltpu.sync_copy(acc.at[pl.ds(r0, _RPT)], out_hbm.at[c, pl.ds(r0, _RPT)])

    return k(ht_flat, gidx2, sdst2, zeros)


def _emit_table(hn, wcat_ref, bcat_ref, out_ref):
    dot = jnp.dot(hn, wcat_ref[...], preferred_element_type=jnp.float32)
    for k in range(_K8):
        out_ref[pl.ds(k * _NB, _NB), :] = (
            dot[:, k * _HALF:(k + 1) * _HALF] + bcat_ref[k])


def _tc_prologue(h, Wcat, bcat):
    def body(h_ref, wcat_ref, bcat_ref, out_ref):
        _emit_table(h_ref[...], wcat_ref, bcat_ref, out_ref)

    return pl.pallas_call(
        body,
        grid=(_GRID,),
        in_specs=[
            pl.BlockSpec((_NB, _OUT), lambda i: (i, 0)),
            pl.BlockSpec((_OUT, _K8 * _HALF), lambda i: (0, 0)),
            pl.BlockSpec((_K8, _HALF), lambda i: (0, 0)),
        ],
        out_specs=pl.BlockSpec((_K8 * _NB, _HALF), lambda i: (i, 0)),
        out_shape=jax.ShapeDtypeStruct((_K8 * _N, _HALF), jnp.float32),
    )(h, Wcat, bcat)


def _gru(a2_ref, h_ref, wrz_ref, wn_ref, brz_ref, bn_ref):
    h = h_ref[...]
    ah = jnp.concatenate([a2_ref[0], a2_ref[1], h], axis=-1)
    rz = jax.nn.sigmoid(
        jnp.dot(ah, wrz_ref[...], preferred_element_type=jnp.float32)
        + brz_ref[...])
    gn = (jnp.dot(ah, wn_ref[...], preferred_element_type=jnp.float32)
          + bn_ref[...])
    r = rz[:, :_OUT]
    z = rz[:, _OUT:]
    n = jnp.tanh(gn[:, :_OUT] + r * gn[:, _OUT:])
    return (1.0 - z) * n + z * h


def _tc_step(a2, h, Wrz, Wn, brz, bn, Wcat, bcat):
    def body(a2_ref, h_ref, wrz_ref, wn_ref, brz_ref, bn_ref,
             wcat_ref, bcat_ref, h_out, ht_out):
        hn = _gru(a2_ref, h_ref, wrz_ref, wn_ref, brz_ref, bn_ref)
        h_out[...] = hn
        _emit_table(hn, wcat_ref, bcat_ref, ht_out)

    return pl.pallas_call(
        body,
        grid=(_GRID,),
        in_specs=[
            pl.BlockSpec((_NCORES, _NB, _HALF), lambda i: (0, i, 0)),
            pl.BlockSpec((_NB, _OUT), lambda i: (i, 0)),
            pl.BlockSpec((2 * _OUT, 2 * _OUT), lambda i: (0, 0)),
            pl.BlockSpec((2 * _OUT, 2 * _OUT), lambda i: (0, 0)),
            pl.BlockSpec((1, 2 * _OUT), lambda i: (0, 0)),
            pl.BlockSpec((1, 2 * _OUT), lambda i: (0, 0)),
            pl.BlockSpec((_OUT, _K8 * _HALF), lambda i: (0, 0)),
            pl.BlockSpec((_K8, _HALF), lambda i: (0, 0)),
        ],
        out_specs=[
            pl.BlockSpec((_NB, _OUT), lambda i: (i, 0)),
            pl.BlockSpec((_K8 * _NB, _HALF), lambda i: (i, 0)),
        ],
        out_shape=[
            jax.ShapeDtypeStruct((_N, _OUT), jnp.float32),
            jax.ShapeDtypeStruct((_K8 * _N, _HALF), jnp.float32),
        ],
    )(a2, h, Wrz, Wn, brz, bn, Wcat, bcat)


def _tc_final(a2, h, h1, Wrz, Wn, brz, bn, clf_wt, clf_b2):
    def body(a2_ref, h_ref, h1_ref, wrz_ref, wn_ref, brz_ref, bn_ref,
             cw_ref, cb_ref, feats_out, res_out):
        i = pl.program_id(0)
        hn = _gru(a2_ref, h_ref, wrz_ref, wn_ref, brz_ref, bn_ref)
        part = jnp.sum(hn + h1_ref[...], axis=0, keepdims=True)

        @pl.when(i == 0)
        def _():
            feats_out[...] = jnp.zeros_like(feats_out)

        feats_out[...] += part

        @pl.when(i == _GRID - 1)
        def _():
            res_out[...] = (
                jnp.dot(feats_out[...], cw_ref[...],
                        preferred_element_type=jnp.float32) + cb_ref[...])

    return pl.pallas_call(
        body,
        grid=(_GRID,),
        in_specs=[
            pl.BlockSpec((_NCORES, _NB, _HALF), lambda i: (0, i, 0)),
            pl.BlockSpec((_NB, _OUT), lambda i: (i, 0)),
            pl.BlockSpec((_NB, _OUT), lambda i: (i, 0)),
            pl.BlockSpec((2 * _OUT, 2 * _OUT), lambda i: (0, 0)),
            pl.BlockSpec((2 * _OUT, 2 * _OUT), lambda i: (0, 0)),
            pl.BlockSpec((1, 2 * _OUT), lambda i: (0, 0)),
            pl.BlockSpec((1, 2 * _OUT), lambda i: (0, 0)),
            pl.BlockSpec((_OUT, 2), lambda i: (0, 0)),
            pl.BlockSpec((1, 2), lambda i: (0, 0)),
        ],
        out_specs=[
            pl.BlockSpec((1, _OUT), lambda i: (0, 0)),
            pl.BlockSpec((1, 2), lambda i: (0, 0)),
        ],
        out_shape=[
            jax.ShapeDtypeStruct((1, _OUT), jnp.float32),
            jax.ShapeDtypeStruct((1, 2), jnp.float32),
        ],
    )(a2, h, h1, Wrz, Wn, brz, bn, clf_wt, clf_b2)


def kernel(x, edge_index, etype, W_msg, b_msg, gru_w_ih, gru_w_hh,
           gru_b_ih, gru_b_hh, clf_w, clf_b):
    src = edge_index[0]
    dst = edge_index[1]

    blk = src // _NB
    off = src % _NB
    gidx = (blk * _K8 + etype) * _NB + off
    pad = _EP - _E
    gidx_p = jnp.concatenate([gidx, jnp.zeros((pad,), jnp.int32)])
    dst_p = jnp.concatenate([dst, jnp.full((pad,), _N, jnp.int32)])
    gidx2 = jnp.stack([gidx_p, gidx_p + _T * _NB]).reshape(
        _NCORES, _NCH, _CHUNK)
    sdst2 = dst_p.reshape(_NCH, _CHUNK)
    zeros = jnp.zeros((_NP, _HALF), jnp.float32)

    WT = jnp.transpose(W_msg, (0, 2, 1)).reshape(_T, _OUT, _NCORES, _HALF)
    Wcat = jnp.transpose(WT, (2, 0, 1, 3)).reshape(_K8 * _OUT, _HALF)
    Wcat = Wcat.reshape(_NCORES * _T, _OUT, _HALF).transpose(1, 0, 2)
    Wcat = Wcat.reshape(_OUT, _K8 * _HALF)
    bcat = b_msg.reshape(_T, _NCORES, _HALF).transpose(1, 0, 2).reshape(
        _K8, _HALF)

    wih = gru_w_ih.T
    whh = gru_w_hh.T
    Wrz = jnp.concatenate([wih[:, :2 * _OUT], whh[:, :2 * _OUT]], axis=0)
    Wn = jnp.concatenate([
        jnp.concatenate([wih[:, 2 * _OUT:],
                         jnp.zeros((_OUT, _OUT), jnp.float32)], axis=1),
        jnp.concatenate([jnp.zeros((_OUT, _OUT), jnp.float32),
                         whh[:, 2 * _OUT:]], axis=1)], axis=0)
    brz = (gru_b_ih[:2 * _OUT] + gru_b_hh[:2 * _OUT]).reshape(1, 2 * _OUT)
    bn = jnp.concatenate([gru_b_ih[2 * _OUT:],
                          gru_b_hh[2 * _OUT:]]).reshape(1, 2 * _OUT)
    clf_wt = clf_w.T
    clf_b2 = clf_b.reshape(1, 2)

    h1 = x
    if x.shape[1] < _OUT:
        h1 = jnp.concatenate(
            [x, jnp.zeros((x.shape[0], _OUT - x.shape[1]), x.dtype)], axis=-1)
    h = h1

    ht = _tc_prologue(h, Wcat, bcat)
    res = None
    for s_i in range(_STEPS):
        a2 = _sc_aggregate(ht, gidx2, sdst2, zeros)
        if s_i < _STEPS - 1:
            h, ht = _tc_step(a2, h, Wrz, Wn, brz, bn, Wcat, bcat)
        else:
            _, res = _tc_final(a2, h, h1, Wrz, Wn, brz, bn, clf_wt, clf_b2)
    return res[0]


# speedup vs baseline: 1.3572x; 1.3572x over previous
import functools

import jax
import jax.numpy as jnp
from jax import lax
from jax.experimental import pallas as pl
from jax.experimental.pallas import tpu as pltpu
from jax.experimental.pallas import tpu_sc as plsc

_N = 50000
_E = 800000
_OUT = 64
_HALF = 32
_T = 4
_STEPS = 8
_NCORES = 2
_NTILES = 16
_CHUNK = 128
_KB = 4
_CPT = 392
_G = _CPT // _KB
_NCH = _CPT * _NTILES
_EP = _NCH * _CHUNK
_NP = 50048
_RPT = _NP // _NTILES
_NB = 2000
_GRID = _N // _NB
_K8 = _NCORES * _T


def _sc_aggregate(ht_flat, gidx2, sdst2, zeros):
    mesh = plsc.VectorSubcoreMesh(
        core_axis_name="c", subcore_axis_name="s",
        num_cores=_NCORES, num_subcores=_NTILES)

    @functools.partial(
        pl.kernel,
        out_type=jax.ShapeDtypeStruct((_NCORES, _NP, 4 * _HALF), jnp.float32),
        mesh=mesh,
        scratch_types=[
            pltpu.VMEM((_KB, _CHUNK), jnp.int32),
            pltpu.VMEM((_KB, _CHUNK), jnp.int32),
            pltpu.VMEM((_KB, _CHUNK, _HALF), jnp.float32),
            pltpu.VMEM_SHARED((_NP, _HALF), jnp.float32),
            pltpu.SemaphoreType.DMA,
        ],
        compiler_params=pltpu.CompilerParams(use_tc_tiling_on_sc=False),
    )
    def k(ht_hbm, gidx_hbm, sdst_hbm, zeros_hbm, out_hbm,
          gidx_v, sidx_v, rows_v, acc, sem):
        c = lax.axis_index("c")
        s = lax.axis_index("s")
        r0 = s * _RPT
        pltpu.sync_copy(zeros_hbm.at[pl.ds(r0, _RPT)], acc.at[pl.ds(r0, _RPT)])
        plsc.subcore_barrier()

        def group(g, carry):
            base = s * _CPT + g * _KB
            pltpu.sync_copy(gidx_hbm.at[c, pl.ds(base, _KB)], gidx_v)
            pltpu.sync_copy(sdst_hbm.at[pl.ds(base, _KB)], sidx_v)
            descs = [pltpu.async_copy(ht_hbm.at[gidx_v.at[j]], rows_v.at[j], sem)
                     for j in range(_KB)]
            for j in range(_KB):
                descs[j].wait()
                pltpu.sync_copy(rows_v.at[j], acc.at[sidx_v.at[j]], add=True)
            return carry

        lax.fori_loop(0, _G, group, 0)
        plsc.subcore_barrier()
        pltpu.sync_copy(acc.at[pl.ds(r0, _RPT)],
                        out_hbm.at[c, pl.ds(r0, _RPT), pl.ds(0, _HALF)])

    return k(ht_flat, gidx2, sdst2, zeros)


def _emit_table(hn, wcat_ref, bcat_ref, out_ref):
    dot = jnp.dot(hn, wcat_ref[...], preferred_element_type=jnp.float32)
    for k in range(_K8):
        plane = dot[:, k * _HALF:(k + 1) * _HALF] + bcat_ref[k]
        out_ref[pl.ds(k * _NB, _NB), pl.ds(0, _HALF)] = plane


def _tc_prologue(h, Wcat, bcat):
    def body(h_ref, wcat_ref, bcat_ref, out_ref):
        _emit_table(h_ref[...], wcat_ref, bcat_ref, out_ref)

    return pl.pallas_call(
        body,
        grid=(_GRID,),
        in_specs=[
            pl.BlockSpec((_NB, _OUT), lambda i: (i, 0)),
            pl.BlockSpec((_OUT, _K8 * _HALF), lambda i: (0, 0)),
            pl.BlockSpec((_K8, _HALF), lambda i: (0, 0)),
        ],
        out_specs=pl.BlockSpec((_K8 * _NB, 4 * _HALF), lambda i: (i, 0)),
        out_shape=jax.ShapeDtypeStruct((_K8 * _N, 4 * _HALF), jnp.float32),
    )(h, Wcat, bcat)


def _gru(a2_ref, h_ref, wrz_ref, wn_ref, brz_ref, bn_ref):
    h = h_ref[...]
    a_lo = a2_ref[0][:, :_HALF]
    a_hi = a2_ref[1][:, :_HALF]
    ah = jnp.concatenate([a_lo, a_hi, h], axis=-1)
    rz = jax.nn.sigmoid(
        jnp.dot(ah, wrz_ref[...], preferred_element_type=jnp.float32)
        + brz_ref[...])
    gn = (jnp.dot(ah, wn_ref[...], preferred_element_type=jnp.float32)
          + bn_ref[...])
    r = rz[:, :_OUT]
    z = rz[:, _OUT:]
    n = jnp.tanh(gn[:, :_OUT] + r * gn[:, _OUT:])
    return (1.0 - z) * n + z * h


def _tc_step(a2, h, Wrz, Wn, brz, bn, Wcat, bcat):
    def body(a2_ref, h_ref, wrz_ref, wn_ref, brz_ref, bn_ref,
             wcat_ref, bcat_ref, h_out, ht_out):
        hn = _gru(a2_ref, h_ref, wrz_ref, wn_ref, brz_ref, bn_ref)
        h_out[...] = hn
        _emit_table(hn, wcat_ref, bcat_ref, ht_out)

    return pl.pallas_call(
        body,
        grid=(_GRID,),
        in_specs=[
            pl.BlockSpec((_NCORES, _NB, 4 * _HALF), lambda i: (0, i, 0)),
            pl.BlockSpec((_NB, _OUT), lambda i: (i, 0)),
            pl.BlockSpec((2 * _OUT, 2 * _OUT), lambda i: (0, 0)),
            pl.BlockSpec((2 * _OUT, 2 * _OUT), lambda i: (0, 0)),
            pl.BlockSpec((1, 2 * _OUT), lambda i: (0, 0)),
            pl.BlockSpec((1, 2 * _OUT), lambda i: (0, 0)),
            pl.BlockSpec((_OUT, _K8 * _HALF), lambda i: (0, 0)),
            pl.BlockSpec((_K8, _HALF), lambda i: (0, 0)),
        ],
        out_specs=[
            pl.BlockSpec((_NB, _OUT), lambda i: (i, 0)),
            pl.BlockSpec((_K8 * _NB, 4 * _HALF), lambda i: (i, 0)),
        ],
        out_shape=[
            jax.ShapeDtypeStruct((_N, _OUT), jnp.float32),
            jax.ShapeDtypeStruct((_K8 * _N, 4 * _HALF), jnp.float32),
        ],
    )(a2, h, Wrz, Wn, brz, bn, Wcat, bcat)


def _tc_final(a2, h, h1, Wrz, Wn, brz, bn, clf_wt, clf_b2):
    def body(a2_ref, h_ref, h1_ref, wrz_ref, wn_ref, brz_ref, bn_ref,
             cw_ref, cb_ref, feats_out, res_out):
        i = pl.program_id(0)
        hn = _gru(a2_ref, h_ref, wrz_ref, wn_ref, brz_ref, bn_ref)
        part = jnp.sum(hn + h1_ref[...], axis=0, keepdims=True)

        @pl.when(i == 0)
        def _():
            feats_out[...] = jnp.zeros_like(feats_out)

        feats_out[...] += part

        @pl.when(i == _GRID - 1)
        def _():
            res_out[...] = (
                jnp.dot(feats_out[...], cw_ref[...],
                        preferred_element_type=jnp.float32) + cb_ref[...])

    return pl.pallas_call(
        body,
        grid=(_GRID,),
        in_specs=[
            pl.BlockSpec((_NCORES, _NB, 4 * _HALF), lambda i: (0, i, 0)),
            pl.BlockSpec((_NB, _OUT), lambda i: (i, 0)),
            pl.BlockSpec((_NB, _OUT), lambda i: (i, 0)),
            pl.BlockSpec((2 * _OUT, 2 * _OUT), lambda i: (0, 0)),
            pl.BlockSpec((2 * _OUT, 2 * _OUT), lambda i: (0, 0)),
            pl.BlockSpec((1, 2 * _OUT), lambda i: (0, 0)),
            pl.BlockSpec((1, 2 * _OUT), lambda i: (0, 0)),
            pl.BlockSpec((_OUT, 2), lambda i: (0, 0)),
            pl.BlockSpec((1, 2), lambda i: (0, 0)),
        ],
        out_specs=[
            pl.BlockSpec((1, _OUT), lambda i: (0, 0)),
            pl.BlockSpec((1, 2), lambda i: (0, 0)),
        ],
        out_shape=[
            jax.ShapeDtypeStruct((1, _OUT), jnp.float32),
            jax.ShapeDtypeStruct((1, 2), jnp.float32),
        ],
    )(a2, h, h1, Wrz, Wn, brz, bn, clf_wt, clf_b2)


def kernel(x, edge_index, etype, W_msg, b_msg, gru_w_ih, gru_w_hh,
           gru_b_ih, gru_b_hh, clf_w, clf_b):
    src = edge_index[0]
    dst = edge_index[1]

    blk = src // _NB
    off = src % _NB
    gidx = ((blk * _K8 + etype) * _NB + off) * 4
    pad = _EP - _E
    gidx_p = jnp.concatenate([gidx, jnp.zeros((pad,), jnp.int32)])
    dst_p = jnp.concatenate([dst, jnp.full((pad,), _N, jnp.int32)])
    gidx2 = jnp.stack([gidx_p, gidx_p + 4 * _T * _NB]).reshape(
        _NCORES, _NCH, _CHUNK)
    sdst2 = dst_p.reshape(_NCH, _CHUNK)
    zeros = jnp.zeros((_NP, _HALF), jnp.float32)

    WT = jnp.transpose(W_msg, (0, 2, 1)).reshape(_T, _OUT, _NCORES, _HALF)
    Wcat = jnp.transpose(WT, (2, 0, 1, 3)).reshape(_K8 * _OUT, _HALF)
    Wcat = Wcat.reshape(_NCORES * _T, _OUT, _HALF).transpose(1, 0, 2)
    Wcat = Wcat.reshape(_OUT, _K8 * _HALF)
    bcat = b_msg.reshape(_T, _NCORES, _HALF).transpose(1, 0, 2).reshape(
        _K8, _HALF)

    wih = gru_w_ih.T
    whh = gru_w_hh.T
    Wrz = jnp.concatenate([wih[:, :2 * _OUT], whh[:, :2 * _OUT]], axis=0)
    Wn = jnp.concatenate([
        jnp.concatenate([wih[:, 2 * _OUT:],
                         jnp.zeros((_OUT, _OUT), jnp.float32)], axis=1),
        jnp.concatenate([jnp.zeros((_OUT, _OUT), jnp.float32),
                         whh[:, 2 * _OUT:]], axis=1)], axis=0)
    brz = (gru_b_ih[:2 * _OUT] + gru_b_hh[:2 * _OUT]).reshape(1, 2 * _OUT)
    bn = jnp.concatenate([gru_b_ih[2 * _OUT:],
                          gru_b_hh[2 * _OUT:]]).reshape(1, 2 * _OUT)
    clf_wt = clf_w.T
    clf_b2 = clf_b.reshape(1, 2)

    h1 = x
    if x.shape[1] < _OUT:
        h1 = jnp.concatenate(
            [x, jnp.zeros((x.shape[0], _OUT - x.shape[1]), x.dtype)], axis=-1)
    h = h1

    ht_wide = _tc_prologue(h, Wcat, bcat)
    res = None
    for s_i in range(_STEPS):
        a2 = _sc_aggregate(ht_wide.reshape(4 * _K8 * _N, _HALF),
                           gidx2, sdst2, zeros)
        if s_i < _STEPS - 1:
            h, ht_wide = _tc_step(a2, h, Wrz, Wn, brz, bn, Wcat, bcat)
        else:
            _, res = _tc_final(a2, h, h1, Wrz, Wn, brz, bn, clf_wt, clf_b2)
    return res[0]


# speedup vs baseline: 1.6776x; 1.2361x over previous
import functools

import jax
import jax.numpy as jnp
from jax import lax
from jax.experimental import pallas as pl
from jax.experimental.pallas import tpu as pltpu
from jax.experimental.pallas import tpu_sc as plsc

_N = 50000
_E = 800000
_OUT = 64
_HALF = 32
_T = 4
_STEPS = 8
_NCORES = 2
_NTILES = 16
_CHUNK = 128
_KB = 3
_CPT = 393
_G = _CPT // _KB
_NCH = _CPT * _NTILES
_EP = _NCH * _CHUNK
_NP = 50048
_RPT = _NP // _NTILES
_NB = 2000
_GRID = _N // _NB
_K8 = _NCORES * _T


def _sc_aggregate(ht_flat, gidx2, sdst2, zeros):
    mesh = plsc.VectorSubcoreMesh(
        core_axis_name="c", subcore_axis_name="s",
        num_cores=_NCORES, num_subcores=_NTILES)

    @functools.partial(
        pl.kernel,
        out_type=jax.ShapeDtypeStruct((_NCORES, _NP, 4 * _HALF), jnp.float32),
        mesh=mesh,
        scratch_types=[
            pltpu.VMEM((2, _KB, _CHUNK), jnp.int32),
            pltpu.VMEM((2, _KB, _CHUNK), jnp.int32),
            pltpu.VMEM((2, _KB, _CHUNK, _HALF), jnp.float32),
            pltpu.VMEM_SHARED((_NP, _HALF), jnp.float32),
            pltpu.SemaphoreType.DMA,
            pltpu.SemaphoreType.DMA,
        ],
        compiler_params=pltpu.CompilerParams(use_tc_tiling_on_sc=False),
    )
    def k(ht_hbm, gidx_hbm, sdst_hbm, zeros_hbm, out_hbm,
          gidx_v, sidx_v, rows_v, acc, sem, sem_i):
        c = lax.axis_index("c")
        s = lax.axis_index("s")
        r0 = s * _RPT
        pltpu.sync_copy(zeros_hbm.at[pl.ds(r0, _RPT)], acc.at[pl.ds(r0, _RPT)])
        plsc.subcore_barrier()

        def fire_gathers(p):
            for j in range(_KB):
                pltpu.async_copy(ht_hbm.at[gidx_v.at[p, j]],
                                 rows_v.at[p, j], sem)

        def fire_idx(g, p):
            base = s * _CPT + g * _KB
            pltpu.async_copy(gidx_hbm.at[c, pl.ds(base, _KB)],
                             gidx_v.at[p], sem_i)
            pltpu.async_copy(sdst_hbm.at[pl.ds(base, _KB)],
                             sidx_v.at[p], sem_i)

        pltpu.sync_copy(gidx_hbm.at[c, pl.ds(s * _CPT, _KB)], gidx_v.at[0])
        pltpu.sync_copy(sdst_hbm.at[pl.ds(s * _CPT, _KB)], sidx_v.at[0])
        fire_gathers(0)
        fire_idx(1, 1)

        def group(g, carry):
            p = lax.rem(g, 2)
            q = lax.rem(g + 1, 2)
            pltpu.make_async_copy(
                gidx_hbm.at[c, pl.ds(0, _KB)], gidx_v.at[q], sem_i).wait()
            pltpu.make_async_copy(
                sdst_hbm.at[pl.ds(0, _KB)], sidx_v.at[q], sem_i).wait()

            @pl.when(g + 1 < _G)
            def _():
                fire_gathers(q)

            for j in range(_KB):
                pltpu.make_async_copy(
                    ht_hbm.at[gidx_v.at[p, j]], rows_v.at[p, j], sem).wait()
                pltpu.sync_copy(rows_v.at[p, j], acc.at[sidx_v.at[p, j]],
                                add=True)
            fire_idx(lax.min(g + 2, _G - 1), p)
            return carry

        lax.fori_loop(0, _G, group, 0)
        pltpu.make_async_copy(
            gidx_hbm.at[c, pl.ds(0, _KB)], gidx_v.at[0], sem_i).wait()
        pltpu.make_async_copy(
            sdst_hbm.at[pl.ds(0, _KB)], sidx_v.at[0], sem_i).wait()
        plsc.subcore_barrier()
        pltpu.sync_copy(acc.at[pl.ds(r0, _RPT)],
                        out_hbm.at[c, pl.ds(r0, _RPT), pl.ds(0, _HALF)])

    return k(ht_flat, gidx2, sdst2, zeros)


def _emit_table(hn, wcat_ref, bcat_ref, out_ref):
    dot = jnp.dot(hn, wcat_ref[...], preferred_element_type=jnp.float32)
    for k in range(_K8):
        plane = dot[:, k * _HALF:(k + 1) * _HALF] + bcat_ref[k]
        out_ref[pl.ds(k * _NB, _NB), pl.ds(0, _HALF)] = plane


def _tc_prologue(h, Wcat, bcat):
    def body(h_ref, wcat_ref, bcat_ref, out_ref):
        _emit_table(h_ref[...], wcat_ref, bcat_ref, out_ref)

    return pl.pallas_call(
        body,
        grid=(_GRID,),
        in_specs=[
            pl.BlockSpec((_NB, _OUT), lambda i: (i, 0)),
            pl.BlockSpec((_OUT, _K8 * _HALF), lambda i: (0, 0)),
            pl.BlockSpec((_K8, _HALF), lambda i: (0, 0)),
        ],
        out_specs=pl.BlockSpec((_K8 * _NB, 4 * _HALF), lambda i: (i, 0)),
        out_shape=jax.ShapeDtypeStruct((_K8 * _N, 4 * _HALF), jnp.float32),
    )(h, Wcat, bcat)


def _gru(a2_ref, h_ref, wrz_ref, wn_ref, brz_ref, bn_ref):
    h = h_ref[...]
    a_lo = a2_ref[0][:, :_HALF]
    a_hi = a2_ref[1][:, :_HALF]
    ah = jnp.concatenate([a_lo, a_hi, h], axis=-1)
    rz = jax.nn.sigmoid(
        jnp.dot(ah, wrz_ref[...], preferred_element_type=jnp.float32)
        + brz_ref[...])
    gn = (jnp.dot(ah, wn_ref[...], preferred_element_type=jnp.float32)
          + bn_ref[...])
    r = rz[:, :_OUT]
    z = rz[:, _OUT:]
    n = jnp.tanh(gn[:, :_OUT] + r * gn[:, _OUT:])
    return (1.0 - z) * n + z * h


def _tc_step(a2, h, Wrz, Wn, brz, bn, Wcat, bcat):
    def body(a2_ref, h_ref, wrz_ref, wn_ref, brz_ref, bn_ref,
             wcat_ref, bcat_ref, h_out, ht_out):
        hn = _gru(a2_ref, h_ref, wrz_ref, wn_ref, brz_ref, bn_ref)
        h_out[...] = hn
        _emit_table(hn, wcat_ref, bcat_ref, ht_out)

    return pl.pallas_call(
        body,
        grid=(_GRID,),
        in_specs=[
            pl.BlockSpec((_NCORES, _NB, 4 * _HALF), lambda i: (0, i, 0)),
            pl.BlockSpec((_NB, _OUT), lambda i: (i, 0)),
            pl.BlockSpec((2 * _OUT, 2 * _OUT), lambda i: (0, 0)),
            pl.BlockSpec((2 * _OUT, 2 * _OUT), lambda i: (0, 0)),
            pl.BlockSpec((1, 2 * _OUT), lambda i: (0, 0)),
            pl.BlockSpec((1, 2 * _OUT), lambda i: (0, 0)),
            pl.BlockSpec((_OUT, _K8 * _HALF), lambda i: (0, 0)),
            pl.BlockSpec((_K8, _HALF), lambda i: (0, 0)),
        ],
        out_specs=[
            pl.BlockSpec((_NB, _OUT), lambda i: (i, 0)),
            pl.BlockSpec((_K8 * _NB, 4 * _HALF), lambda i: (i, 0)),
        ],
        out_shape=[
            jax.ShapeDtypeStruct((_N, _OUT), jnp.float32),
            jax.ShapeDtypeStruct((_K8 * _N, 4 * _HALF), jnp.float32),
        ],
    )(a2, h, Wrz, Wn, brz, bn, Wcat, bcat)


def _tc_final(a2, h, h1, Wrz, Wn, brz, bn, clf_wt, clf_b2):
    def body(a2_ref, h_ref, h1_ref, wrz_ref, wn_ref, brz_ref, bn_ref,
             cw_ref, cb_ref, feats_out, res_out):
        i = pl.program_id(0)
        hn = _gru(a2_ref, h_ref, wrz_ref, wn_ref, brz_ref, bn_ref)
        part = jnp.sum(hn + h1_ref[...], axis=0, keepdims=True)

        @pl.when(i == 0)
        def _():
            feats_out[...] = jnp.zeros_like(feats_out)

        feats_out[...] += part

        @pl.when(i == _GRID - 1)
        def _():
            res_out[...] = (
                jnp.dot(feats_out[...], cw_ref[...],
                        preferred_element_type=jnp.float32) + cb_ref[...])

    return pl.pallas_call(
        body,
        grid=(_GRID,),
        in_specs=[
            pl.BlockSpec((_NCORES, _NB, 4 * _HALF), lambda i: (0, i, 0)),
            pl.BlockSpec((_NB, _OUT), lambda i: (i, 0)),
            pl.BlockSpec((_NB, _OUT), lambda i: (i, 0)),
            pl.BlockSpec((2 * _OUT, 2 * _OUT), lambda i: (0, 0)),
            pl.BlockSpec((2 * _OUT, 2 * _OUT), lambda i: (0, 0)),
            pl.BlockSpec((1, 2 * _OUT), lambda i: (0, 0)),
            pl.BlockSpec((1, 2 * _OUT), lambda i: (0, 0)),
            pl.BlockSpec((_OUT, 2), lambda i: (0, 0)),
            pl.BlockSpec((1, 2), lambda i: (0, 0)),
        ],
        out_specs=[
            pl.BlockSpec((1, _OUT), lambda i: (0, 0)),
            pl.BlockSpec((1, 2), lambda i: (0, 0)),
        ],
        out_shape=[
            jax.ShapeDtypeStruct((1, _OUT), jnp.float32),
            jax.ShapeDtypeStruct((1, 2), jnp.float32),
        ],
    )(a2, h, h1, Wrz, Wn, brz, bn, clf_wt, clf_b2)


def kernel(x, edge_index, etype, W_msg, b_msg, gru_w_ih, gru_w_hh,
           gru_b_ih, gru_b_hh, clf_w, clf_b):
    src = edge_index[0]
    dst = edge_index[1]

    blk = src // _NB
    off = src % _NB
    gidx = ((blk * _K8 + etype) * _NB + off) * 4
    pad = _EP - _E
    gidx_p = jnp.concatenate([gidx, jnp.zeros((pad,), jnp.int32)])
    dst_p = jnp.concatenate([dst, jnp.full((pad,), _N, jnp.int32)])
    gidx2 = jnp.stack([gidx_p, gidx_p + 4 * _T * _NB]).reshape(
        _NCORES, _NCH, _CHUNK)
    sdst2 = dst_p.reshape(_NCH, _CHUNK)
    zeros = jnp.zeros((_NP, _HALF), jnp.float32)

    WT = jnp.transpose(W_msg, (0, 2, 1)).reshape(_T, _OUT, _NCORES, _HALF)
    Wcat = jnp.transpose(WT, (2, 0, 1, 3)).reshape(_K8 * _OUT, _HALF)
    Wcat = Wcat.reshape(_NCORES * _T, _OUT, _HALF).transpose(1, 0, 2)
    Wcat = Wcat.reshape(_OUT, _K8 * _HALF)
    bcat = b_msg.reshape(_T, _NCORES, _HALF).transpose(1, 0, 2).reshape(
        _K8, _HALF)

    wih = gru_w_ih.T
    whh = gru_w_hh.T
    Wrz = jnp.concatenate([wih[:, :2 * _OUT], whh[:, :2 * _OUT]], axis=0)
    Wn = jnp.concatenate([
        jnp.concatenate([wih[:, 2 * _OUT:],
                         jnp.zeros((_OUT, _OUT), jnp.float32)], axis=1),
        jnp.concatenate([jnp.zeros((_OUT, _OUT), jnp.float32),
                         whh[:, 2 * _OUT:]], axis=1)], axis=0)
    brz = (gru_b_ih[:2 * _OUT] + gru_b_hh[:2 * _OUT]).reshape(1, 2 * _OUT)
    bn = jnp.concatenate([gru_b_ih[2 * _OUT:],
                          gru_b_hh[2 * _OUT:]]).reshape(1, 2 * _OUT)
    clf_wt = clf_w.T
    clf_b2 = clf_b.reshape(1, 2)

    h1 = x
    if x.shape[1] < _OUT:
        h1 = jnp.concatenate(
            [x, jnp.zeros((x.shape[0], _OUT - x.shape[1]), x.dtype)], axis=-1)
    h = h1

    ht_wide = _tc_prologue(h, Wcat, bcat)
    res = None
    for s_i in range(_STEPS):
        a2 = _sc_aggregate(ht_wide.reshape(4 * _K8 * _N, _HALF),
                           gidx2, sdst2, zeros)
        if s_i < _STEPS - 1:
            h, ht_wide = _tc_step(a2, h, Wrz, Wn, brz, bn, Wcat, bcat)
        else:
            _, res = _tc_final(a2, h, h1, Wrz, Wn, brz, bn, clf_wt, clf_b2)
    return res[0]


# speedup vs baseline: 2.0812x; 1.2406x over previous
import functools

import jax
import jax.numpy as jnp
from jax import lax
from jax.experimental import pallas as pl
from jax.experimental.pallas import tpu as pltpu
from jax.experimental.pallas import tpu_sc as plsc

_N = 50000
_E = 800000
_OUT = 64
_HALF = 32
_T = 4
_STEPS = 8
_NCORES = 2
_NTILES = 16
_CHUNK = 128
_KB = 3
_CPT = 393
_G = _CPT // _KB
_NCH = _CPT * _NTILES
_EP = _NCH * _CHUNK
_NP = 50048
_RPT = _NP // _NTILES
_NB = 2000
_GRID = _N // _NB
_K8 = _NCORES * _T


def _sc_aggregate(ht_flat, gidx2, sdst2, zeros):
    mesh = plsc.VectorSubcoreMesh(
        core_axis_name="c", subcore_axis_name="s",
        num_cores=_NCORES, num_subcores=_NTILES)

    @functools.partial(
        pl.kernel,
        out_type=jax.ShapeDtypeStruct((_NP, 4 * _HALF), jnp.float32),
        mesh=mesh,
        scratch_types=[
            pltpu.VMEM((2, _KB, _CHUNK), jnp.int32),
            pltpu.VMEM((2, _KB, _CHUNK), jnp.int32),
            pltpu.VMEM((2, _KB, _CHUNK, _HALF), jnp.float32),
            pltpu.VMEM_SHARED((_NP, _HALF), jnp.float32),
            pltpu.SemaphoreType.DMA,
            pltpu.SemaphoreType.DMA,
            pltpu.SemaphoreType.DMA,
        ],
        compiler_params=pltpu.CompilerParams(use_tc_tiling_on_sc=False),
    )
    def k(ht_hbm, gidx_hbm, sdst_hbm, zeros_hbm, out_hbm,
          gidx_v, sidx_v, rows_v, acc, sem, sem_i, sem_s):
        c = lax.axis_index("c")
        s = lax.axis_index("s")
        r0 = s * _RPT
        pltpu.sync_copy(zeros_hbm.at[pl.ds(r0, _RPT)], acc.at[pl.ds(r0, _RPT)])
        plsc.subcore_barrier()

        def fire_gathers(p):
            for j in range(_KB):
                pltpu.async_copy(ht_hbm.at[gidx_v.at[p, j]],
                                 rows_v.at[p, j], sem)

        def fire_idx(g, p):
            base = s * _CPT + g * _KB
            pltpu.async_copy(gidx_hbm.at[c, pl.ds(base, _KB)],
                             gidx_v.at[p], sem_i)
            pltpu.async_copy(sdst_hbm.at[pl.ds(base, _KB)],
                             sidx_v.at[p], sem_i)

        pltpu.sync_copy(gidx_hbm.at[c, pl.ds(s * _CPT, _KB)], gidx_v.at[0])
        pltpu.sync_copy(sdst_hbm.at[pl.ds(s * _CPT, _KB)], sidx_v.at[0])
        fire_gathers(0)
        fire_idx(1, 1)

        def drain_scatters(p):
            for j in range(_KB):
                pltpu.make_async_copy(
                    ht_hbm.at[gidx_v.at[p, j]], rows_v.at[p, j], sem_s).wait()

        def group(g, carry):
            p = lax.rem(g, 2)
            q = lax.rem(g + 1, 2)
            pltpu.make_async_copy(
                gidx_hbm.at[c, pl.ds(0, _KB)], gidx_v.at[q], sem_i).wait()
            pltpu.make_async_copy(
                sdst_hbm.at[pl.ds(0, _KB)], sidx_v.at[q], sem_i).wait()

            @pl.when(g > 0)
            def _():
                drain_scatters(q)

            @pl.when(g + 1 < _G)
            def _():
                fire_gathers(q)

            for j in range(_KB):
                pltpu.make_async_copy(
                    ht_hbm.at[gidx_v.at[p, j]], rows_v.at[p, j], sem).wait()
                pltpu.async_copy(rows_v.at[p, j], acc.at[sidx_v.at[p, j]],
                                 sem_s, add=True)
            fire_idx(lax.min(g + 2, _G - 1), p)
            return carry

        lax.fori_loop(0, _G, group, 0)
        drain_scatters((_G - 1) % 2)
        pltpu.make_async_copy(
            gidx_hbm.at[c, pl.ds(0, _KB)], gidx_v.at[0], sem_i).wait()
        pltpu.make_async_copy(
            sdst_hbm.at[pl.ds(0, _KB)], sidx_v.at[0], sem_i).wait()
        plsc.subcore_barrier()
        pltpu.sync_copy(acc.at[pl.ds(r0, _RPT)],
                        out_hbm.at[pl.ds(r0, _RPT), pl.ds(c * _HALF, _HALF)])

    return k(ht_flat, gidx2, sdst2, zeros)


def _emit_table(hn, wcat_ref, bcat_ref, out_ref):
    dot = jnp.dot(hn, wcat_ref[...], preferred_element_type=jnp.float32)
    for c in range(_NCORES):
        for t in range(_T):
            k = c * _T + t
            plane = dot[:, k * _HALF:(k + 1) * _HALF] + bcat_ref[k]
            out_ref[pl.ds(c * _NB, _NB), pl.ds(t * _HALF, _HALF)] = plane


def _tc_prologue(h, Wcat, bcat):
    def body(h_ref, wcat_ref, bcat_ref, out_ref):
        _emit_table(h_ref[...], wcat_ref, bcat_ref, out_ref)

    return pl.pallas_call(
        body,
        grid=(_GRID,),
        in_specs=[
            pl.BlockSpec((_NB, _OUT), lambda i: (i, 0)),
            pl.BlockSpec((_OUT, _K8 * _HALF), lambda i: (0, 0)),
            pl.BlockSpec((_K8, _HALF), lambda i: (0, 0)),
        ],
        out_specs=pl.BlockSpec((_NCORES * _NB, 4 * _HALF), lambda i: (i, 0)),
        out_shape=jax.ShapeDtypeStruct((_NCORES * _N, 4 * _HALF), jnp.float32),
    )(h, Wcat, bcat)


def _gru(a2_ref, h_ref, wrz_ref, wn_ref, brz_ref, bn_ref):
    h = h_ref[...]
    ah = jnp.concatenate([a2_ref[:, :2 * _HALF], h], axis=-1)
    rz = jax.nn.sigmoid(
        jnp.dot(ah, wrz_ref[...], preferred_element_type=jnp.float32)
        + brz_ref[...])
    gn = (jnp.dot(ah, wn_ref[...], preferred_element_type=jnp.float32)
          + bn_ref[...])
    r = rz[:, :_OUT]
    z = rz[:, _OUT:]
    n = jnp.tanh(gn[:, :_OUT] + r * gn[:, _OUT:])
    return (1.0 - z) * n + z * h


def _tc_step(a2, h, Wrz, Wn, brz, bn, Wcat, bcat):
    def body(a2_ref, h_ref, wrz_ref, wn_ref, brz_ref, bn_ref,
             wcat_ref, bcat_ref, h_out, ht_out):
        hn = _gru(a2_ref, h_ref, wrz_ref, wn_ref, brz_ref, bn_ref)
        h_out[...] = hn
        _emit_table(hn, wcat_ref, bcat_ref, ht_out)

    return pl.pallas_call(
        body,
        grid=(_GRID,),
        in_specs=[
            pl.BlockSpec((_NB, 4 * _HALF), lambda i: (i, 0)),
            pl.BlockSpec((_NB, _OUT), lambda i: (i, 0)),
            pl.BlockSpec((2 * _OUT, 2 * _OUT), lambda i: (0, 0)),
            pl.BlockSpec((2 * _OUT, 2 * _OUT), lambda i: (0, 0)),
            pl.BlockSpec((1, 2 * _OUT), lambda i: (0, 0)),
            pl.BlockSpec((1, 2 * _OUT), lambda i: (0, 0)),
            pl.BlockSpec((_OUT, _K8 * _HALF), lambda i: (0, 0)),
            pl.BlockSpec((_K8, _HALF), lambda i: (0, 0)),
        ],
        out_specs=[
            pl.BlockSpec((_NB, _OUT), lambda i: (i, 0)),
            pl.BlockSpec((_NCORES * _NB, 4 * _HALF), lambda i: (i, 0)),
        ],
        out_shape=[
            jax.ShapeDtypeStruct((_N, _OUT), jnp.float32),
            jax.ShapeDtypeStruct((_NCORES * _N, 4 * _HALF), jnp.float32),
        ],
    )(a2, h, Wrz, Wn, brz, bn, Wcat, bcat)


def _tc_final(a2, h, h1, Wrz, Wn, brz, bn, clf_wt, clf_b2):
    def body(a2_ref, h_ref, h1_ref, wrz_ref, wn_ref, brz_ref, bn_ref,
             cw_ref, cb_ref, feats_out, res_out):
        i = pl.program_id(0)
        hn = _gru(a2_ref, h_ref, wrz_ref, wn_ref, brz_ref, bn_ref)
        part = jnp.sum(hn + h1_ref[...], axis=0, keepdims=True)

        @pl.when(i == 0)
        def _():
            feats_out[...] = jnp.zeros_like(feats_out)

        feats_out[...] += part

        @pl.when(i == _GRID - 1)
        def _():
            res_out[...] = (
                jnp.dot(feats_out[...], cw_ref[...],
                        preferred_element_type=jnp.float32) + cb_ref[...])

    return pl.pallas_call(
        body,
        grid=(_GRID,),
        in_specs=[
            pl.BlockSpec((_NB, 4 * _HALF), lambda i: (i, 0)),
            pl.BlockSpec((_NB, _OUT), lambda i: (i, 0)),
            pl.BlockSpec((_NB, _OUT), lambda i: (i, 0)),
            pl.BlockSpec((2 * _OUT, 2 * _OUT), lambda i: (0, 0)),
            pl.BlockSpec((2 * _OUT, 2 * _OUT), lambda i: (0, 0)),
            pl.BlockSpec((1, 2 * _OUT), lambda i: (0, 0)),
            pl.BlockSpec((1, 2 * _OUT), lambda i: (0, 0)),
            pl.BlockSpec((_OUT, 2), lambda i: (0, 0)),
            pl.BlockSpec((1, 2), lambda i: (0, 0)),
        ],
        out_specs=[
            pl.BlockSpec((1, _OUT), lambda i: (0, 0)),
            pl.BlockSpec((1, 2), lambda i: (0, 0)),
        ],
        out_shape=[
            jax.ShapeDtypeStruct((1, _OUT), jnp.float32),
            jax.ShapeDtypeStruct((1, 2), jnp.float32),
        ],
    )(a2, h, h1, Wrz, Wn, brz, bn, clf_wt, clf_b2)


def kernel(x, edge_index, etype, W_msg, b_msg, gru_w_ih, gru_w_hh,
           gru_b_ih, gru_b_hh, clf_w, clf_b):
    src = edge_index[0]
    dst = edge_index[1]

    blk = src // _NB
    off = src % _NB
    gidx = (blk * (2 * _NB) + off) * 4 + etype
    pad = _EP - _E
    gidx_p = jnp.concatenate([gidx, jnp.zeros((pad,), jnp.int32)])
    dst_p = jnp.concatenate([dst, jnp.full((pad,), _N, jnp.int32)])
    gidx2 = jnp.stack([gidx_p, gidx_p + 4 * _NB]).reshape(
        _NCORES, _NCH, _CHUNK)
    sdst2 = dst_p.reshape(_NCH, _CHUNK)
    zeros = jnp.zeros((_NP, _HALF), jnp.float32)

    WT = jnp.transpose(W_msg, (0, 2, 1)).reshape(_T, _OUT, _NCORES, _HALF)
    Wcat = jnp.transpose(WT, (2, 0, 1, 3)).reshape(_K8 * _OUT, _HALF)
    Wcat = Wcat.reshape(_NCORES * _T, _OUT, _HALF).transpose(1, 0, 2)
    Wcat = Wcat.reshape(_OUT, _K8 * _HALF)
    bcat = b_msg.reshape(_T, _NCORES, _HALF).transpose(1, 0, 2).reshape(
        _K8, _HALF)

    wih = gru_w_ih.T
    whh = gru_w_hh.T
    Wrz = jnp.concatenate([wih[:, :2 * _OUT], whh[:, :2 * _OUT]], axis=0)
    Wn = jnp.concatenate([
        jnp.concatenate([wih[:, 2 * _OUT:],
                         jnp.zeros((_OUT, _OUT), jnp.float32)], axis=1),
        jnp.concatenate([jnp.zeros((_OUT, _OUT), jnp.float32),
                         whh[:, 2 * _OUT:]], axis=1)], axis=0)
    brz = (gru_b_ih[:2 * _OUT] + gru_b_hh[:2 * _OUT]).reshape(1, 2 * _OUT)
    bn = jnp.concatenate([gru_b_ih[2 * _OUT:],
                          gru_b_hh[2 * _OUT:]]).reshape(1, 2 * _OUT)
    clf_wt = clf_w.T
    clf_b2 = clf_b.reshape(1, 2)

    h1 = x
    if x.shape[1] < _OUT:
        h1 = jnp.concatenate(
            [x, jnp.zeros((x.shape[0], _OUT - x.shape[1]), x.dtype)], axis=-1)
    h = h1

    ht_wide = _tc_prologue(h, Wcat, bcat)
    res = None
    for s_i in range(_STEPS):
        a2 = _sc_aggregate(ht_wide.reshape(4 * _NCORES * _N, _HALF),
                           gidx2, sdst2, zeros)
        if s_i < _STEPS - 1:
            h, ht_wide = _tc_step(a2, h, Wrz, Wn, brz, bn, Wcat, bcat)
        else:
            _, res = _tc_final(a2, h, h1, Wrz, Wn, brz, bn, clf_wt, clf_b2)
    return res[0]
